# Initial kernel scaffold; baseline (speedup 1.0000x reference)
import functools

import jax
import jax.numpy as jnp
from jax import lax
from jax.experimental import pallas as pl
from jax.experimental.pallas import tpu as pltpu
from jax.experimental.pallas import tpu_sc as plsc

NV = 10000
EV = 320000
NE = 320000
ET = 640000

NC = 2
NS = 16
LANES = 16
NW = NC * NS
CH = 128
JUNK = 16

_i32 = jnp.int32
_f32 = jnp.float32

_MESH = plsc.VectorSubcoreMesh(core_axis_name="c", subcore_axis_name="s")


def _lane_iota():
  return lax.iota(_i32, LANES)


def _elem(vec, j):
  return jnp.sum(jnp.where(_lane_iota() == j, vec, 0))


def _sc_agg_full(n, d, e_pad):
  assert e_pad % (CH * NW) == 0 and n % NS == 0
  chunks_per_worker = e_pad // (CH * NW)
  rpt = n // NS

  @functools.partial(
      pl.kernel,
      out_type=jax.ShapeDtypeStruct((2 * n, d), _f32),
      mesh=_MESH,
      scratch_types=[
          pltpu.VMEM((CH,), _i32),
          pltpu.VMEM((CH,), _i32),
          pltpu.VMEM((CH, d), _f32),
          pltpu.VMEM_SHARED((n + JUNK, d), _f32),
          pltpu.SemaphoreType.DMA,
      ],
  )
  def k(g_hbm, src_hbm, dst_hbm, out_hbm, sidx_v, didx_v, rows_v, acc, sem):
    c = lax.axis_index("c")
    t = lax.axis_index("s")
    w = c * NS + t
    pltpu.sync_copy(g_hbm.at[pl.ds(t * rpt, rpt)], acc.at[pl.ds(t * rpt, rpt)])
    plsc.subcore_barrier()

    def chunk(i, carry):
      off = (i * NW + w) * CH
      pltpu.sync_copy(src_hbm.at[pl.ds(off, CH)], sidx_v)
      pltpu.sync_copy(dst_hbm.at[pl.ds(off, CH)], didx_v)
      pltpu.async_copy(g_hbm.at[sidx_v], rows_v, sem).wait()
      pltpu.sync_copy(rows_v, acc.at[didx_v], add=True)
      return carry

    lax.fori_loop(0, chunks_per_worker, chunk, 0)
    plsc.subcore_barrier()
    pltpu.sync_copy(acc.at[pl.ds(t * rpt, rpt)],
                    out_hbm.at[pl.ds(c * n + t * rpt, rpt)])

  return k


def _sc_deg_full(n, e_pad):
  assert e_pad % (CH * NW) == 0 and n % NS == 0
  chunks_per_worker = e_pad // (CH * NW)
  rpt = n // NS
  n_init = -(-rpt // CH)

  @functools.partial(
      pl.kernel,
      out_type=jax.ShapeDtypeStruct((2 * n, LANES), _f32),
      mesh=_MESH,
      scratch_types=[
          pltpu.VMEM((CH,), _i32),
          pltpu.VMEM((CH, LANES), _f32),
          pltpu.VMEM_SHARED((n + JUNK, LANES), _f32),
      ],
  )
  def k(dst_hbm, ones_hbm, out_hbm, didx_v, ones_v, acc):
    c = lax.axis_index("c")
    t = lax.axis_index("s")
    w = c * NS + t
    pltpu.sync_copy(ones_hbm, ones_v)
    for r in range(n_init):
      pltpu.sync_copy(ones_v, acc.at[pl.ds(t * rpt + r * CH, CH)])
    plsc.subcore_barrier()

    def chunk(i, carry):
      off = (i * NW + w) * CH
      pltpu.sync_copy(dst_hbm.at[pl.ds(off, CH)], didx_v)
      pltpu.sync_copy(ones_v, acc.at[didx_v], add=True)
      return carry

    lax.fori_loop(0, chunks_per_worker, chunk, 0)
    plsc.subcore_barrier()
    pltpu.sync_copy(acc.at[pl.ds(t * rpt, rpt)],
                    out_hbm.at[pl.ds(c * n + t * rpt, rpt)])

  return k


def _sc_agg_binned(n, d, r, p, with_gather):
  assert n == r * p and p % NC == 0 and r % NS == 0
  rpt = r // NS
  ppc = p // NC
  width = d if with_gather else LANES
  n_init = -(-rpt // CH)

  scratch = [
      pltpu.VMEM((CH,), _i32),
      pltpu.VMEM((CH,), _i32),
      pltpu.VMEM((CH, width), _f32),
      pltpu.VMEM((64,), _i32),
      pltpu.VMEM_SHARED((r + JUNK, width), _f32),
      pltpu.SemaphoreType.DMA,
  ]

  def body(*refs):
    if with_gather:
      (g_hbm, se_hbm, de_hbm, bnd_hbm, out_hbm,
       sidx_v, didx_v, rows_v, bnd_v, acc, sem) = refs
    else:
      (se_hbm, de_hbm, bnd_hbm, ones_hbm, out_hbm,
       sidx_v, didx_v, rows_v, bnd_v, acc, sem) = refs
    c = lax.axis_index("c")
    t = lax.axis_index("s")
    w = c * NS + t
    iot = _lane_iota()
    pltpu.sync_copy(bnd_hbm, bnd_v)
    if not with_gather:
      pltpu.sync_copy(ones_hbm, rows_v)

    def one_pass(j, carry):
      pw = c * ppc + j
      base = pw * r
      bv = bnd_v[pl.ds(pw, LANES)]
      start = _elem(bv, 0)
      end = _elem(bv, 1)
      if with_gather:
        pltpu.sync_copy(g_hbm.at[pl.ds(base + t * rpt, rpt)],
                        acc.at[pl.ds(t * rpt, rpt)])
      else:
        for rr in range(n_init):
          pltpu.sync_copy(rows_v, acc.at[pl.ds(t * rpt + rr * CH, CH)])
      plsc.subcore_barrier()

      start_al = (start // LANES) * LANES
      trip = (end - start_al + CH - 1) // CH
      n_t = (trip - t + NS - 1) // NS

      def chunk(kk, carry2):
        off = start_al + (kk * NS + t) * CH
        pltpu.sync_copy(de_hbm.at[pl.ds(off, CH)], didx_v)
        if with_gather:
          pltpu.sync_copy(se_hbm.at[pl.ds(off, CH)], sidx_v)
        for jj in range(CH // LANES):
          ge = off + jj * LANES + iot
          valid = (ge >= start) & (ge < end)
          sl = pl.ds(jj * LANES, LANES)
          didx_v[sl] = jnp.where(valid, didx_v[sl], r + iot)
          if with_gather:
            sidx_v[sl] = jnp.where(valid, sidx_v[sl], w * LANES + iot)
        if with_gather:
          pltpu.async_copy(g_hbm.at[sidx_v], rows_v, sem).wait()
        pltpu.sync_copy(rows_v, acc.at[didx_v], add=True)
        return carry2

      lax.fori_loop(0, n_t, chunk, 0)
      plsc.subcore_barrier()
      pltpu.sync_copy(acc.at[pl.ds(t * rpt, rpt)],
                      out_hbm.at[pl.ds(base + t * rpt, rpt)])
      plsc.subcore_barrier()
      return carry

    lax.fori_loop(0, ppc, one_pass, 0)

  return functools.partial(
      pl.kernel,
      out_type=jax.ShapeDtypeStruct((n, width), _f32),
      mesh=_MESH,
      scratch_types=scratch,
  )(body)


_BR = 2000


def _tc(body, n, dout, in_arrays, in_shapes):
  specs = []
  for s in in_shapes:
    if s[0] is None:
      specs.append(pl.BlockSpec(s[1], lambda i: (0, 0)))
    else:
      specs.append(pl.BlockSpec((_BR, s[1]), lambda i: (i, 0)))
  return pl.pallas_call(
      body,
      grid=(n // _BR,),
      in_specs=specs,
      out_specs=pl.BlockSpec((_BR, dout), lambda i: (i, 0)),
      out_shape=jax.ShapeDtypeStruct((n, dout), _f32),
  )(*in_arrays)


def _mm(x, w, dinv):
  n, kdim = x.shape
  dout = w.shape[1]

  def body(x_ref, w_ref, d_ref, o_ref):
    o_ref[...] = jnp.dot(x_ref[...], w_ref[...],
                         preferred_element_type=_f32) * d_ref[...]

  return _tc(body, n, dout, (x, w, dinv),
             ((0, kdim), (None, (kdim, dout)), (0, 1)))


def _post(a, dinv, b):
  n, d = a.shape

  def body(a_ref, d_ref, b_ref, o_ref):
    o_ref[...] = jnp.maximum(a_ref[...] * d_ref[...] + b_ref[...], 0.0)

  return _tc(body, n, d, (a, dinv, b), ((0, d), (0, 1), (None, (1, d))))


def _postv(a0, a1, g, dinv, b):
  n, d = a0.shape

  def body(a0_ref, a1_ref, g_ref, d_ref, b_ref, o_ref):
    acc = a0_ref[...] + a1_ref[...] - g_ref[...]
    o_ref[...] = jnp.maximum(acc * d_ref[...] + b_ref[...], 0.0)

  return _tc(body, n, d, (a0, a1, g, dinv, b),
             ((0, d), (0, d), (0, d), (0, 1), (None, (1, d))))


def _post_mm(a, dinv, b, w2):
  n, d = a.shape
  dout = w2.shape[1]

  def body(a_ref, d_ref, b_ref, w_ref, o_ref):
    h = jnp.maximum(a_ref[...] * d_ref[...] + b_ref[...], 0.0)
    o_ref[...] = jnp.dot(h, w_ref[...],
                         preferred_element_type=_f32) * d_ref[...]

  return _tc(body, n, dout, (a, dinv, b, w2),
             ((0, d), (0, 1), (None, (1, d)), (None, (d, dout))))


def _postv_mm(a0, a1, g, dinv, b, w2):
  n, d = a0.shape
  dout = w2.shape[1]

  def body(a0_ref, a1_ref, g_ref, d_ref, b_ref, w_ref, o_ref):
    acc = a0_ref[...] + a1_ref[...] - g_ref[...]
    h = jnp.maximum(acc * d_ref[...] + b_ref[...], 0.0)
    o_ref[...] = jnp.dot(h, w_ref[...],
                         preferred_element_type=_f32) * d_ref[...]

  return _tc(body, n, dout, (a0, a1, g, dinv, b, w2),
             ((0, d), (0, d), (0, d), (0, 1), (None, (1, d)),
              (None, (d, dout))))


def _out2(a_first, b_first, a_second, b_second, dinv):
  n, d1 = a_first.shape
  d2 = a_second.shape[1]

  def body(a1_ref, b1_ref, a2_ref, b2_ref, d_ref, o_ref):
    y1 = jnp.maximum(a1_ref[...] * d_ref[...] + b1_ref[...], 0.0)
    y2 = jnp.maximum(a2_ref[...] * d_ref[...] + b2_ref[...], 0.0)
    o_ref[...] = jnp.concatenate([y1, y2], axis=1)

  return _tc(body, n, d1 + d2, (a_first, b_first, a_second, b_second, dinv),
             ((0, d1), (None, (1, d1)), (0, d2), (None, (1, d2)), (0, 1)))


def _out2v(a0f, a1f, gf, bf, a0s, a1s, gs, bs, dinv):
  n, d1 = a0f.shape
  d2 = a0s.shape[1]

  def body(a0f_r, a1f_r, gf_r, bf_r, a0s_r, a1s_r, gs_r, bs_r, d_ref, o_ref):
    y1 = jnp.maximum((a0f_r[...] + a1f_r[...] - gf_r[...]) * d_ref[...]
                     + bf_r[...], 0.0)
    y2 = jnp.maximum((a0s_r[...] + a1s_r[...] - gs_r[...]) * d_ref[...]
                     + bs_r[...], 0.0)
    o_ref[...] = jnp.concatenate([y1, y2], axis=1)

  return _tc(body, n, d1 + d2,
             (a0f, a1f, gf, bf, a0s, a1s, gs, bs, dinv),
             ((0, d1), (0, d1), (0, d1), (None, (1, d1)),
              (0, d2), (0, d2), (0, d2), (None, (1, d2)), (0, 1)))


def _dinv1(deg):
  n = deg.shape[0]

  def body(deg_ref, o_ref):
    o_ref[...] = lax.rsqrt(jnp.maximum(deg_ref[...][:, :1], 1e-12))

  return _tc(body, n, 1, (deg,), ((0, LANES),))


def _dinv2(deg0, deg1):
  n = deg0.shape[0]

  def body(d0_ref, d1_ref, o_ref):
    deg = d0_ref[...][:, :1] + d1_ref[...][:, :1] - 1.0
    o_ref[...] = lax.rsqrt(jnp.maximum(deg, 1e-12))

  return _tc(body, n, 1, (deg0, deg1), ((0, LANES), (0, LANES)))


def kernel(feature_v, edge_index, feature_e, trans_edge_index,
           W1v, b1v, W1e, b1e, Ws1, bs1, Ws2, bs2,
           W2v, b2v, W3v, b3v, W2e, b2e, W3e, b3e):
  ei = edge_index.astype(_i32)
  te = trans_edge_index.astype(_i32)
  src_v, dst_v = ei[0], ei[1]
  src_e, dst_e = te[0], te[1]

  evp = -(-EV // (CH * NW)) * (CH * NW)
  padn = evp - EV
  pad_ids = jnp.arange(padn, dtype=_i32)
  srcs_vp = jnp.concatenate([src_v, pad_ids % 512])
  dsts_vp = jnp.concatenate([dst_v, NV + (pad_ids % JUNK)])

  r = 10000
  p = NE // r
  bucket = dst_e // r
  order = jnp.argsort(bucket)
  se = src_e[order]
  dst_s = dst_e[order]
  de = dst_s - (dst_s // r) * r
  bnd = jnp.searchsorted(dst_s // r,
                         jnp.arange(p + 1, dtype=_i32)).astype(_i32)
  bnd64 = jnp.concatenate([bnd, jnp.full((64 - (p + 1),), ET, _i32)])
  se_p = jnp.concatenate([se, jnp.zeros((256,), _i32)])
  de_p = jnp.concatenate([de, jnp.full((256,), r, _i32)])
  ones_sc = jnp.ones((CH, LANES), _f32)

  b1v_ = b1v.reshape(1, -1)
  b1e_ = b1e.reshape(1, -1)
  bs1_ = bs1.reshape(1, -1)
  bs2_ = bs2.reshape(1, -1)
  b2v_ = b2v.reshape(1, -1)
  b3v_ = b3v.reshape(1, -1)
  b2e_ = b2e.reshape(1, -1)
  b3e_ = b3e.reshape(1, -1)

  agg_v = _sc_agg_full(NV, 128, evp)
  deg_v_k = _sc_deg_full(NV, evp)
  agg_e128 = _sc_agg_binned(NE, 128, r, p, True)
  agg_e64 = _sc_agg_binned(NE, 64, r, p, True)
  deg_e_k = _sc_agg_binned(NE, 0, r, p, False)

  degv = deg_v_k(dsts_vp, ones_sc)
  dege = deg_e_k(se_p, de_p, bnd64, ones_sc)
  dinv_v = _dinv2(degv[:NV], degv[NV:])
  dinv_e = _dinv1(dege)

  g1 = _mm(feature_v, W1v, dinv_v)
  A1 = agg_v(g1, srcs_vp, dsts_vp)
  fv = _postv(A1[:NV], A1[NV:], g1, dinv_v, b1v_)

  gs1 = _mm(fv, Ws1, dinv_v)
  As1 = agg_v(gs1, srcs_vp, dsts_vp)
  gs2 = _postv_mm(As1[:NV], As1[NV:], gs1, dinv_v, bs1_, Ws2)
  As2 = agg_v(gs2, srcs_vp, dsts_vp)

  g2 = _mm(fv, W2v, dinv_v)
  A2 = agg_v(g2, srcs_vp, dsts_vp)
  g3 = _postv_mm(A2[:NV], A2[NV:], g2, dinv_v, b2v_, W3v)
  A3 = agg_v(g3, srcs_vp, dsts_vp)

  fv_out = _out2v(A3[:NV], A3[NV:], g3, b3v_,
                  As2[:NV], As2[NV:], gs2, bs2_, dinv_v)

  ge1 = _mm(feature_e, W1e, dinv_e)
  Ae1 = agg_e128(ge1, se_p, de_p, bnd64)
  fe = _post(Ae1, dinv_e, b1e_)

  ges1 = _mm(fe, Ws1, dinv_e)
  Aes1 = agg_e128(ges1, se_p, de_p, bnd64)
  ges2 = _post_mm(Aes1, dinv_e, bs1_, Ws2)
  Aes2 = agg_e128(ges2, se_p, de_p, bnd64)

  ge2 = _mm(fe, W2e, dinv_e)
  Ae2 = agg_e64(ge2, se_p, de_p, bnd64)
  ge3 = _post_mm(Ae2, dinv_e, b2e_, W3e)
  Ae3 = agg_e64(ge3, se_p, de_p, bnd64)

  fe_out = _out2(Ae3, b3e_, Aes2, bs2_, dinv_e)

  return fv_out, fe_out


# speedup vs baseline: 10.5103x; 10.5103x over previous
import functools

import jax
import jax.numpy as jnp
from jax import lax
from jax.experimental import pallas as pl
from jax.experimental.pallas import tpu as pltpu
from jax.experimental.pallas import tpu_sc as plsc

NV = 10000
EV = 320000
NE = 320000
ET = 640000

NC = 2
NS = 16
LANES = 16
NW = NC * NS
CH = 128
JUNK = 16

_i32 = jnp.int32
_f32 = jnp.float32

_MESH = plsc.VectorSubcoreMesh(core_axis_name="c", subcore_axis_name="s")


def _lane_iota():
  return lax.iota(_i32, LANES)


STEP = 624
SZ = 640


def _slab(t):
  return pl.ds(t * STEP, SZ)


def _sc_agg_full(n, d, e_pad):
  assert e_pad % (CH * NW) == 0 and STEP * (NS - 1) + SZ == n
  chunks_per_worker = e_pad // (CH * NW)

  @functools.partial(
      pl.kernel,
      out_type=jax.ShapeDtypeStruct((2 * n, d), _f32),
      mesh=_MESH,
      scratch_types=[
          pltpu.VMEM((CH,), _i32),
          pltpu.VMEM((CH,), _i32),
          pltpu.VMEM((CH, d), _f32),
          pltpu.VMEM_SHARED((n + JUNK, d), _f32),
          pltpu.SemaphoreType.DMA,
      ],
  )
  def k(g_hbm, src_hbm, dst_hbm, out_hbm, sidx_v, didx_v, rows_v, acc, sem):
    c = lax.axis_index("c")
    t = lax.axis_index("s")
    w = c * NS + t
    pltpu.sync_copy(g_hbm.at[_slab(t)], acc.at[_slab(t)])
    plsc.subcore_barrier()

    def chunk(i, carry):
      off = (i * NW + w) * CH
      pltpu.sync_copy(src_hbm.at[pl.ds(off, CH)], sidx_v)
      pltpu.sync_copy(dst_hbm.at[pl.ds(off, CH)], didx_v)
      pltpu.async_copy(g_hbm.at[sidx_v], rows_v, sem).wait()
      pltpu.sync_copy(rows_v, acc.at[didx_v], add=True)
      return carry

    lax.fori_loop(0, chunks_per_worker, chunk, 0)
    plsc.subcore_barrier()
    pltpu.sync_copy(acc.at[_slab(t)],
                    out_hbm.at[pl.ds(c * n + t * STEP, SZ)])

  return k


def _sc_deg_full(n, e_pad):
  assert e_pad % (CH * NW) == 0 and STEP * (NS - 1) + SZ == n
  chunks_per_worker = e_pad // (CH * NW)
  n_init = SZ // CH

  @functools.partial(
      pl.kernel,
      out_type=jax.ShapeDtypeStruct((2 * n, 128), _f32),
      mesh=_MESH,
      scratch_types=[
          pltpu.VMEM((CH,), _i32),
          pltpu.VMEM((CH, 128), _f32),
          pltpu.VMEM_SHARED((n + JUNK, 128), _f32),
      ],
  )
  def k(dst_hbm, ones_hbm, out_hbm, didx_v, ones_v, acc):
    c = lax.axis_index("c")
    t = lax.axis_index("s")
    w = c * NS + t
    pltpu.sync_copy(ones_hbm, ones_v)
    for r in range(n_init):
      pltpu.sync_copy(ones_v, acc.at[pl.ds(t * STEP + r * CH, CH)])
    plsc.subcore_barrier()

    def chunk(i, carry):
      off = (i * NW + w) * CH
      pltpu.sync_copy(dst_hbm.at[pl.ds(off, CH)], didx_v)
      pltpu.sync_copy(ones_v, acc.at[didx_v], add=True)
      return carry

    lax.fori_loop(0, chunks_per_worker, chunk, 0)
    plsc.subcore_barrier()
    pltpu.sync_copy(acc.at[_slab(t)],
                    out_hbm.at[pl.ds(c * n + t * STEP, SZ)])

  return k


def _sc_agg_binned(n, d, r, p, with_gather):
  assert n == r * p and p % NC == 0 and STEP * (NS - 1) + SZ == r
  ppc = p // NC
  width = d if with_gather else 128
  n_init = SZ // CH

  scratch = [
      pltpu.VMEM((CH,), _i32),
      pltpu.VMEM((CH,), _i32),
      pltpu.VMEM((CH, width), _f32),
      pltpu.VMEM((64,), _i32),
      pltpu.VMEM_SHARED((r + JUNK, width), _f32),
      pltpu.SemaphoreType.DMA,
  ]

  def body(*refs):
    if with_gather:
      (g_hbm, se_hbm, de_hbm, bnd_hbm, out_hbm,
       sidx_v, didx_v, rows_v, bnd_v, acc, sem) = refs
    else:
      (se_hbm, de_hbm, bnd_hbm, ones_hbm, out_hbm,
       sidx_v, didx_v, rows_v, bnd_v, acc, sem) = refs
    c = lax.axis_index("c")
    t = lax.axis_index("s")
    w = c * NS + t
    iot = _lane_iota()
    pltpu.sync_copy(bnd_hbm, bnd_v)
    if not with_gather:
      pltpu.sync_copy(ones_hbm, rows_v)

    def one_pass(j, carry):
      pw = c * ppc + j
      base = pw * r
      bv = bnd_v[pl.ds(pw, LANES)]
      start = bv[0]
      end = bv[1]
      if with_gather:
        pltpu.sync_copy(g_hbm.at[pl.ds(base + t * STEP, SZ)],
                        acc.at[_slab(t)])
      else:
        for rr in range(n_init):
          pltpu.sync_copy(rows_v, acc.at[pl.ds(t * STEP + rr * CH, CH)])
      plsc.subcore_barrier()

      start_al = (start // LANES) * LANES
      trip = (end - start_al + CH - 1) // CH
      n_t = (trip - t + NS - 1) // NS

      def chunk(kk, carry2):
        off = start_al + (kk * NS + t) * CH
        pltpu.sync_copy(de_hbm.at[pl.ds(off, CH)], didx_v)
        if with_gather:
          pltpu.sync_copy(se_hbm.at[pl.ds(off, CH)], sidx_v)
        for jj in range(CH // LANES):
          ge = off + jj * LANES + iot
          valid = (ge >= start) & (ge < end)
          sl = pl.ds(jj * LANES, LANES)
          didx_v[sl] = jnp.where(valid, didx_v[sl], r + iot)
          if with_gather:
            sidx_v[sl] = jnp.where(valid, sidx_v[sl], w * LANES + iot)
        if with_gather:
          pltpu.async_copy(g_hbm.at[sidx_v], rows_v, sem).wait()
        pltpu.sync_copy(rows_v, acc.at[didx_v], add=True)
        return carry2

      lax.fori_loop(0, n_t, chunk, 0)
      plsc.subcore_barrier()
      pltpu.sync_copy(acc.at[_slab(t)],
                      out_hbm.at[pl.ds(base + t * STEP, SZ)])
      plsc.subcore_barrier()
      return carry

    lax.fori_loop(0, ppc, one_pass, 0)

  return functools.partial(
      pl.kernel,
      out_type=jax.ShapeDtypeStruct((n, width), _f32),
      mesh=_MESH,
      scratch_types=scratch,
  )(body)


_BR = 2000


def _tc(body, n, dout, in_arrays, in_shapes):
  specs = []
  for s in in_shapes:
    if s[0] is None:
      specs.append(pl.BlockSpec(s[1], lambda i: (0, 0)))
    else:
      specs.append(pl.BlockSpec((_BR, s[1]), lambda i: (i, 0)))
  return pl.pallas_call(
      body,
      grid=(n // _BR,),
      in_specs=specs,
      out_specs=pl.BlockSpec((_BR, dout), lambda i: (i, 0)),
      out_shape=jax.ShapeDtypeStruct((n, dout), _f32),
  )(*in_arrays)


def _mm(x, w, dinv, pad_to=None):
  n, kdim = x.shape
  dout = w.shape[1]
  width = pad_to or dout

  def body(x_ref, w_ref, d_ref, o_ref):
    y = jnp.dot(x_ref[...], w_ref[...],
                preferred_element_type=_f32) * d_ref[...]
    if width > dout:
      y = jnp.concatenate([y, jnp.zeros((y.shape[0], width - dout), _f32)],
                          axis=1)
    o_ref[...] = y

  return _tc(body, n, width, (x, w, dinv),
             ((0, kdim), (None, (kdim, dout)), (0, 1)))


def _post(a, dinv, b):
  n, d = a.shape

  def body(a_ref, d_ref, b_ref, o_ref):
    o_ref[...] = jnp.maximum(a_ref[...] * d_ref[...] + b_ref[...], 0.0)

  return _tc(body, n, d, (a, dinv, b), ((0, d), (0, 1), (None, (1, d))))


def _postv(a0, a1, g, dinv, b):
  n, d = a0.shape

  def body(a0_ref, a1_ref, g_ref, d_ref, b_ref, o_ref):
    acc = a0_ref[...] + a1_ref[...] - g_ref[...]
    o_ref[...] = jnp.maximum(acc * d_ref[...] + b_ref[...], 0.0)

  return _tc(body, n, d, (a0, a1, g, dinv, b),
             ((0, d), (0, d), (0, d), (0, 1), (None, (1, d))))


def _post_mm(a, dinv, b, w2, real_d=None, pad_to=None):
  n, d = a.shape
  rd = real_d or d
  kd = w2.shape[0]
  dout = w2.shape[1]
  width = pad_to or dout

  def body(a_ref, d_ref, b_ref, w_ref, o_ref):
    h = jnp.maximum(a_ref[...][:, :rd] * d_ref[...] + b_ref[...], 0.0)
    y = jnp.dot(h, w_ref[...], preferred_element_type=_f32) * d_ref[...]
    if width > dout:
      y = jnp.concatenate([y, jnp.zeros((y.shape[0], width - dout), _f32)],
                          axis=1)
    o_ref[...] = y

  return _tc(body, n, width, (a, dinv, b, w2),
             ((0, d), (0, 1), (None, (1, rd)), (None, (kd, dout))))


def _postv_mm(a0, a1, g, dinv, b, w2):
  n, d = a0.shape
  dout = w2.shape[1]

  def body(a0_ref, a1_ref, g_ref, d_ref, b_ref, w_ref, o_ref):
    acc = a0_ref[...] + a1_ref[...] - g_ref[...]
    h = jnp.maximum(acc * d_ref[...] + b_ref[...], 0.0)
    o_ref[...] = jnp.dot(h, w_ref[...],
                         preferred_element_type=_f32) * d_ref[...]

  return _tc(body, n, dout, (a0, a1, g, dinv, b, w2),
             ((0, d), (0, d), (0, d), (0, 1), (None, (1, d)),
              (None, (d, dout))))


def _out2(a_first, b_first, a_second, b_second, dinv, real_d1=None):
  n, d1 = a_first.shape
  rd1 = real_d1 or d1
  d2 = a_second.shape[1]

  def body(a1_ref, b1_ref, a2_ref, b2_ref, d_ref, o_ref):
    y1 = jnp.maximum(a1_ref[...][:, :rd1] * d_ref[...] + b1_ref[...], 0.0)
    y2 = jnp.maximum(a2_ref[...] * d_ref[...] + b2_ref[...], 0.0)
    o_ref[...] = jnp.concatenate([y1, y2], axis=1)

  return _tc(body, n, rd1 + d2, (a_first, b_first, a_second, b_second, dinv),
             ((0, d1), (None, (1, rd1)), (0, d2), (None, (1, d2)), (0, 1)))


def _out2v(a0f, a1f, gf, bf, a0s, a1s, gs, bs, dinv):
  n, d1 = a0f.shape
  d2 = a0s.shape[1]

  def body(a0f_r, a1f_r, gf_r, bf_r, a0s_r, a1s_r, gs_r, bs_r, d_ref, o_ref):
    y1 = jnp.maximum((a0f_r[...] + a1f_r[...] - gf_r[...]) * d_ref[...]
                     + bf_r[...], 0.0)
    y2 = jnp.maximum((a0s_r[...] + a1s_r[...] - gs_r[...]) * d_ref[...]
                     + bs_r[...], 0.0)
    o_ref[...] = jnp.concatenate([y1, y2], axis=1)

  return _tc(body, n, d1 + d2,
             (a0f, a1f, gf, bf, a0s, a1s, gs, bs, dinv),
             ((0, d1), (0, d1), (0, d1), (None, (1, d1)),
              (0, d2), (0, d2), (0, d2), (None, (1, d2)), (0, 1)))


def _dinv1(deg):
  n = deg.shape[0]

  def body(deg_ref, o_ref):
    o_ref[...] = lax.rsqrt(jnp.maximum(deg_ref[...][:, :1], 1e-12))

  return _tc(body, n, 1, (deg,), ((0, 128),))


def _dinv2(deg0, deg1):
  n = deg0.shape[0]

  def body(d0_ref, d1_ref, o_ref):
    deg = d0_ref[...][:, :1] + d1_ref[...][:, :1] - 1.0
    o_ref[...] = lax.rsqrt(jnp.maximum(deg, 1e-12))

  return _tc(body, n, 1, (deg0, deg1), ((0, 128), (0, 128)))


def kernel(feature_v, edge_index, feature_e, trans_edge_index,
           W1v, b1v, W1e, b1e, Ws1, bs1, Ws2, bs2,
           W2v, b2v, W3v, b3v, W2e, b2e, W3e, b3e):
  ei = edge_index.astype(_i32)
  te = trans_edge_index.astype(_i32)
  src_v, dst_v = ei[0], ei[1]
  src_e, dst_e = te[0], te[1]

  evp = -(-EV // (CH * NW)) * (CH * NW)
  padn = evp - EV
  pad_ids = jnp.arange(padn, dtype=_i32)
  srcs_vp = jnp.concatenate([src_v, pad_ids % 512])
  dsts_vp = jnp.concatenate([dst_v, NV + (pad_ids % JUNK)])

  r = 10000
  p = NE // r
  bucket = dst_e // r
  order = jnp.argsort(bucket)
  se = src_e[order]
  dst_s = dst_e[order]
  de = dst_s - (dst_s // r) * r
  bnd = jnp.searchsorted(dst_s // r,
                         jnp.arange(p + 1, dtype=_i32)).astype(_i32)
  bnd64 = jnp.concatenate([bnd, jnp.full((64 - (p + 1),), ET, _i32)])
  se_p = jnp.concatenate([se, jnp.zeros((256,), _i32)])
  de_p = jnp.concatenate([de, jnp.full((256,), r, _i32)])
  ones_sc = jnp.ones((CH, 128), _f32)

  b1v_ = b1v.reshape(1, -1)
  b1e_ = b1e.reshape(1, -1)
  bs1_ = bs1.reshape(1, -1)
  bs2_ = bs2.reshape(1, -1)
  b2v_ = b2v.reshape(1, -1)
  b3v_ = b3v.reshape(1, -1)
  b2e_ = b2e.reshape(1, -1)
  b3e_ = b3e.reshape(1, -1)

  agg_v = _sc_agg_full(NV, 128, evp)
  deg_v_k = _sc_deg_full(NV, evp)
  agg_e128 = _sc_agg_binned(NE, 128, r, p, True)
  deg_e_k = _sc_agg_binned(NE, 0, r, p, False)

  degv = deg_v_k(dsts_vp, ones_sc)
  dege = deg_e_k(se_p, de_p, bnd64, ones_sc)
  dinv_v = _dinv2(degv[:NV], degv[NV:])
  dinv_e = _dinv1(dege)

  g1 = _mm(feature_v, W1v, dinv_v)
  A1 = agg_v(g1, srcs_vp, dsts_vp)
  fv = _postv(A1[:NV], A1[NV:], g1, dinv_v, b1v_)

  gs1 = _mm(fv, Ws1, dinv_v)
  As1 = agg_v(gs1, srcs_vp, dsts_vp)
  gs2 = _postv_mm(As1[:NV], As1[NV:], gs1, dinv_v, bs1_, Ws2)
  As2 = agg_v(gs2, srcs_vp, dsts_vp)

  g2 = _mm(fv, W2v, dinv_v)
  A2 = agg_v(g2, srcs_vp, dsts_vp)
  g3 = _postv_mm(A2[:NV], A2[NV:], g2, dinv_v, b2v_, W3v)
  A3 = agg_v(g3, srcs_vp, dsts_vp)

  fv_out = _out2v(A3[:NV], A3[NV:], g3, b3v_,
                  As2[:NV], As2[NV:], gs2, bs2_, dinv_v)

  ge1 = _mm(feature_e, W1e, dinv_e)
  Ae1 = agg_e128(ge1, se_p, de_p, bnd64)
  fe = _post(Ae1, dinv_e, b1e_)

  ges1 = _mm(fe, Ws1, dinv_e)
  Aes1 = agg_e128(ges1, se_p, de_p, bnd64)
  ges2 = _post_mm(Aes1, dinv_e, bs1_, Ws2)
  Aes2 = agg_e128(ges2, se_p, de_p, bnd64)

  ge2 = _mm(fe, W2e, dinv_e, pad_to=128)
  Ae2 = agg_e128(ge2, se_p, de_p, bnd64)
  ge3 = _post_mm(Ae2, dinv_e, b2e_, W3e, real_d=64, pad_to=128)
  Ae3 = agg_e128(ge3, se_p, de_p, bnd64)

  fe_out = _out2(Ae3, b3e_, Aes2, bs2_, dinv_e, real_d1=64)

  return fv_out, fe_out
